# baseline (speedup 1.0000x reference)
import functools

import jax
import jax.numpy as jnp
from jax import lax
from jax.experimental import pallas as pl
from jax.experimental.pallas import tpu as pltpu
from jax.experimental.pallas import tpu_sc as plsc

N = 10000
E = 320000
D_IN = 128
D_H = 128
D_OUT = 64
HW1 = 72
D_AUG = 136

N_PAD = 10048
BN = 2512
GRID = N_PAD // BN
CH = 128
EPW = 10240
E_PAD = 32 * EPW
RPS = N_PAD // 16


def _seg_body(col_split, ch, nbuf, ih, *refs):
    nch = (E_PAD // 16 if col_split else EPW) // ch
    nchh = nch // ih
    ngh = nchh // nbuf
    (p_hbm, src_hbm, dst_hbm, z_hbm, acc_out,
     src_v, dst_v, rows_a, table_sh, acc_sh, sem_i, *sems) = refs
    sem_g = sems[0:nbuf]
    sem_s = sems[nbuf:2 * nbuf]
    core = lax.axis_index("c")
    sid = lax.axis_index("s")
    wid = sid if col_split else sid * 2 + core
    rbase = sid * RPS

    ld1 = pltpu.async_copy(src_hbm.at[pl.ds(wid * nch, nchh)], src_v, sem_i)
    ld2 = pltpu.async_copy(dst_hbm.at[pl.ds(wid * nch, nchh)], dst_v, sem_i)
    if col_split:
        pltpu.sync_copy(p_hbm.at[core, pl.ds(rbase, RPS)],
                        table_sh.at[pl.ds(rbase, RPS)])
    else:
        pltpu.sync_copy(p_hbm.at[pl.ds(rbase, RPS)],
                        table_sh.at[pl.ds(rbase, RPS)])
    pltpu.sync_copy(z_hbm, acc_sh.at[pl.ds(rbase, RPS)])
    ld1.wait()
    ld2.wait()
    plsc.subcore_barrier()

    def gather(j, k):
        pltpu.async_copy(table_sh.at[src_v.at[j]], rows_a.at[k], sem_g[k])

    def wait_gather(j, k):
        pltpu.make_async_copy(table_sh.at[src_v.at[j]], rows_a.at[k],
                              sem_g[k]).wait()

    def scat(j, k):
        pltpu.async_copy(rows_a.at[k], acc_sh.at[dst_v.at[j]], sem_s[k],
                         add=True)

    def wait_scat(j, k):
        pltpu.make_async_copy(rows_a.at[k], acc_sh.at[dst_v.at[j]],
                              sem_s[k]).wait()

    def run_stage():
        for k in range(nbuf):
            gather(k, k)

        def group(j2, _):
            base = j2 * nbuf
            for k in range(nbuf):
                wait_gather(base + k, k)
                scat(base + k, k)
            for k in range(nbuf):
                wait_scat(base + k, k)
                gather(base + nbuf + k, k)
            return _
        lax.fori_loop(0, ngh - 1, group, None)

        last = (ngh - 1) * nbuf
        for k in range(nbuf):
            wait_gather(last + k, k)
            scat(last + k, k)
        for k in range(nbuf):
            wait_scat(last + k, k)

    run_stage()
    for h in range(1, ih):
        pltpu.sync_copy(src_hbm.at[pl.ds(wid * nch + h * nchh, nchh)], src_v)
        pltpu.sync_copy(dst_hbm.at[pl.ds(wid * nch + h * nchh, nchh)], dst_v)
        run_stage()

    plsc.subcore_barrier()

    pltpu.sync_copy(acc_sh.at[pl.ds(rbase, RPS)],
                    acc_out.at[core, pl.ds(rbase, RPS)])


def _make_segsum(d, col_split, ch, nbuf, ih):
    mesh = plsc.VectorSubcoreMesh(core_axis_name="c", subcore_axis_name="s")
    nch = (E_PAD // 16 if col_split else EPW) // ch
    scratch = [
        pltpu.VMEM((nch // ih, ch), jnp.int32),
        pltpu.VMEM((nch // ih, ch), jnp.int32),
        pltpu.VMEM((nbuf, ch, d), jnp.float32),
        pltpu.VMEM_SHARED((N_PAD, d), jnp.float32),
        pltpu.VMEM_SHARED((N_PAD, d), jnp.float32),
    ]
    scratch += [pltpu.SemaphoreType.DMA] * (1 + 2 * nbuf)
    return pl.kernel(
        functools.partial(_seg_body, col_split, ch, nbuf, ih),
        out_type=jax.ShapeDtypeStruct((2, N_PAD, d), jnp.float32),
        mesh=mesh,
        scratch_types=scratch,
        compiler_params=pltpu.CompilerParams(use_tc_tiling_on_sc=False),
        name=f"sage_segsum_d{d}",
    )


CH1 = 80
_segsum1 = _make_segsum(HW1, True, CH1, 4, 4)
_segsum2 = _make_segsum(D_OUT, False, CH, 4, 2)


def _tc1_body(x_ref, w_ref, wrow_ref, t_ref):
    t_ref[0, :, :] = (jnp.dot(x_ref[...], w_ref[0, :, :],
                              preferred_element_type=jnp.float32)
                      + wrow_ref[0, :, :])


_tc1 = pl.pallas_call(
    _tc1_body,
    grid=(2, GRID),
    in_specs=[
        pl.BlockSpec((BN, D_IN), lambda c, i: (i, 0)),
        pl.BlockSpec((1, D_IN, HW1), lambda c, i: (c, 0, 0)),
        pl.BlockSpec((1, 1, HW1), lambda c, i: (c, 0, 0)),
    ],
    out_specs=pl.BlockSpec((1, BN, HW1), lambda c, i: (c, i, 0)),
    out_shape=jax.ShapeDtypeStruct((2, N_PAD, HW1), jnp.float32),
    name="sage_tc1",
)


def _tc2_body(x_ref, acc_ref, ws1_ref, b1_ref, ws2_ref, wn2_ref, b2_ref,
              p2_ref, hws_ref, invd_ref):
    dsum = acc_ref[1, :, D_H - HW1:D_H - HW1 + 1]
    inv = 1.0 / jnp.maximum(dsum, 1.0)
    invd_ref[...] = inv
    base = (jnp.dot(x_ref[...], ws1_ref[...],
                    preferred_element_type=jnp.float32) + b1_ref[...])
    h0 = jnp.maximum(base[:, 0:HW1] + acc_ref[0, :, :] * inv, 0.0)
    h1 = jnp.maximum(base[:, HW1:D_H]
                     + acc_ref[1, :, 0:D_H - HW1] * inv, 0.0)
    p2_ref[...] = (
        jnp.dot(h0, wn2_ref[0:HW1, :], preferred_element_type=jnp.float32)
        + jnp.dot(h1, wn2_ref[HW1:D_H, :], preferred_element_type=jnp.float32))
    hws_ref[...] = (
        jnp.dot(h0, ws2_ref[0:HW1, :], preferred_element_type=jnp.float32)
        + jnp.dot(h1, ws2_ref[HW1:D_H, :], preferred_element_type=jnp.float32)
        + b2_ref[...])


_tc2 = pl.pallas_call(
    _tc2_body,
    grid=(GRID,),
    in_specs=[
        pl.BlockSpec((BN, D_IN), lambda i: (i, 0)),
        pl.BlockSpec((2, BN, HW1), lambda i: (0, i, 0)),
        pl.BlockSpec((D_IN, D_H), lambda i: (0, 0)),
        pl.BlockSpec((1, D_H), lambda i: (0, 0)),
        pl.BlockSpec((D_H, D_OUT), lambda i: (0, 0)),
        pl.BlockSpec((D_H, D_OUT), lambda i: (0, 0)),
        pl.BlockSpec((1, D_OUT), lambda i: (0, 0)),
    ],
    out_specs=[
        pl.BlockSpec((BN, D_OUT), lambda i: (i, 0)),
        pl.BlockSpec((BN, D_OUT), lambda i: (i, 0)),
        pl.BlockSpec((BN, 1), lambda i: (i, 0)),
    ],
    out_shape=[
        jax.ShapeDtypeStruct((N_PAD, D_OUT), jnp.float32),
        jax.ShapeDtypeStruct((N_PAD, D_OUT), jnp.float32),
        jax.ShapeDtypeStruct((N_PAD, 1), jnp.float32),
    ],
    name="sage_tc2",
)


def _tc3_body(hws_ref, acc_ref, invd_ref, out_ref):
    a = acc_ref[0, :, :] + acc_ref[1, :, :]
    out_ref[...] = hws_ref[...] + a * invd_ref[...]


_tc3 = pl.pallas_call(
    _tc3_body,
    grid=(GRID,),
    in_specs=[
        pl.BlockSpec((BN, D_OUT), lambda i: (i, 0)),
        pl.BlockSpec((2, BN, D_OUT), lambda i: (0, i, 0)),
        pl.BlockSpec((BN, 1), lambda i: (i, 0)),
    ],
    out_specs=pl.BlockSpec((BN, D_OUT), lambda i: (i, 0)),
    out_shape=jax.ShapeDtypeStruct((N_PAD, D_OUT), jnp.float32),
    name="sage_tc3",
)


def kernel(x, edge_index, W_self1, W_neigh1, b1, W_self2, W_neigh2, b2):
    src = edge_index[0]
    dst = edge_index[1]
    pad = E_PAD - E
    src_f = jnp.concatenate([src, jnp.zeros((pad,), jnp.int32)])
    dst_f = jnp.concatenate([dst, jnp.full((pad,), N, jnp.int32)])
    src_p = src_f.reshape(-1, CH)
    dst_p = dst_f.reshape(-1, CH)
    src_p1 = src_f.reshape(-1, CH1)
    dst_p1 = dst_f.reshape(-1, CH1)
    x_p = jnp.pad(x, ((0, N_PAD - N), (0, 0)))
    wa0 = W_neigh1[:, 0:HW1]
    wa1 = jnp.zeros((D_IN, HW1), jnp.float32)
    wa1 = wa1.at[:, 0:D_H - HW1].set(W_neigh1[:, HW1:D_H])
    w_aug = jnp.stack([wa0, wa1])
    wrow = jnp.zeros((2, 1, HW1), jnp.float32).at[1, 0, D_H - HW1].set(1.0)

    z1 = jnp.zeros((RPS, HW1), jnp.float32)
    z2 = jnp.zeros((RPS, D_OUT), jnp.float32)

    t1 = _tc1(x_p, w_aug, wrow)
    acc1 = _segsum1(t1, src_p1, dst_p1, z1)
    p2, hws2, invd = _tc2(x_p, acc1, W_self1, b1.reshape(1, D_H),
                          W_self2, W_neigh2, b2.reshape(1, D_OUT))
    acc2 = _segsum2(p2, src_p, dst_p, z2)
    out = _tc3(hws2, acc2, invd)
    return out[:N]


# speedup vs baseline: 1.0632x; 1.0632x over previous
import functools

import jax
import jax.numpy as jnp
from jax import lax
from jax.experimental import pallas as pl
from jax.experimental.pallas import tpu as pltpu
from jax.experimental.pallas import tpu_sc as plsc

N = 10000
E = 320000
D_IN = 128
D_H = 128
D_OUT = 64
HW1 = 72
D_AUG = 136

N_PAD = 10048
BN = 2512
GRID = N_PAD // BN
CH = 128
EPW = 10240
E_PAD = 32 * EPW
RPS = N_PAD // 16


def _seg_body(col_split, ch, nbuf, ih, *refs):
    nch = (E_PAD // 16 if col_split else EPW) // ch
    nchh = nch // ih
    ngh = nchh // nbuf
    (p_hbm, src_hbm, dst_hbm, z_hbm, acc_out,
     src_v, dst_v, rows_a, table_sh, acc_sh, sem_i, *sems) = refs
    sem_g = sems[0:nbuf]
    sem_s = sems[nbuf:2 * nbuf]
    core = lax.axis_index("c")
    sid = lax.axis_index("s")
    wid = sid if col_split else sid * 2 + core
    rbase = sid * RPS

    ld1 = pltpu.async_copy(src_hbm.at[pl.ds(wid * nch, nchh)], src_v, sem_i)
    ld2 = pltpu.async_copy(dst_hbm.at[pl.ds(wid * nch, nchh)], dst_v, sem_i)
    if col_split:
        pltpu.sync_copy(p_hbm.at[core, pl.ds(rbase, RPS)],
                        table_sh.at[pl.ds(rbase, RPS)])
    else:
        pltpu.sync_copy(p_hbm.at[pl.ds(rbase, RPS)],
                        table_sh.at[pl.ds(rbase, RPS)])
    pltpu.sync_copy(z_hbm, acc_sh.at[pl.ds(rbase, RPS)])
    ld1.wait()
    ld2.wait()
    plsc.subcore_barrier()

    def gather(j, k):
        pltpu.async_copy(table_sh.at[src_v.at[j]], rows_a.at[k], sem_g[k])

    def wait_gather(j, k):
        pltpu.make_async_copy(table_sh.at[src_v.at[j]], rows_a.at[k],
                              sem_g[k]).wait()

    def scat(j, k):
        pltpu.async_copy(rows_a.at[k], acc_sh.at[dst_v.at[j]], sem_s[k],
                         add=True)

    def wait_scat(j, k):
        pltpu.make_async_copy(rows_a.at[k], acc_sh.at[dst_v.at[j]],
                              sem_s[k]).wait()

    def run_stage():
        for k in range(nbuf):
            gather(k, k)

        def group(j2, _):
            base = j2 * nbuf
            for k in range(nbuf):
                wait_gather(base + k, k)
                scat(base + k, k)
            for k in range(nbuf):
                wait_scat(base + k, k)
                gather(base + nbuf + k, k)
            return _
        lax.fori_loop(0, ngh - 1, group, None)

        last = (ngh - 1) * nbuf
        for k in range(nbuf):
            wait_gather(last + k, k)
            scat(last + k, k)
        for k in range(nbuf):
            wait_scat(last + k, k)

    run_stage()
    for h in range(1, ih):
        pltpu.sync_copy(src_hbm.at[pl.ds(wid * nch + h * nchh, nchh)], src_v)
        pltpu.sync_copy(dst_hbm.at[pl.ds(wid * nch + h * nchh, nchh)], dst_v)
        run_stage()

    plsc.subcore_barrier()

    pltpu.sync_copy(acc_sh.at[pl.ds(rbase, RPS)],
                    acc_out.at[core, pl.ds(rbase, RPS)])


def _make_segsum(d, col_split, ch, nbuf, ih):
    mesh = plsc.VectorSubcoreMesh(core_axis_name="c", subcore_axis_name="s")
    nch = (E_PAD // 16 if col_split else EPW) // ch
    scratch = [
        pltpu.VMEM((nch // ih, ch), jnp.int32),
        pltpu.VMEM((nch // ih, ch), jnp.int32),
        pltpu.VMEM((nbuf, ch, d), jnp.float32),
        pltpu.VMEM_SHARED((N_PAD, d), jnp.float32),
        pltpu.VMEM_SHARED((N_PAD, d), jnp.float32),
    ]
    scratch += [pltpu.SemaphoreType.DMA] * (1 + 2 * nbuf)
    return pl.kernel(
        functools.partial(_seg_body, col_split, ch, nbuf, ih),
        out_type=jax.ShapeDtypeStruct((2, N_PAD, d), jnp.float32),
        mesh=mesh,
        scratch_types=scratch,
        compiler_params=pltpu.CompilerParams(use_tc_tiling_on_sc=False),
        name=f"sage_segsum_d{d}",
    )


_segsum1 = _make_segsum(HW1, True, CH, 2, 2)
_segsum2 = _make_segsum(D_OUT, False, CH, 4, 2)


def _tc1_body(x_ref, w_ref, wrow_ref, t_ref):
    t_ref[0, :, :] = (jnp.dot(x_ref[...], w_ref[0, :, :],
                              preferred_element_type=jnp.float32)
                      + wrow_ref[0, :, :])


_tc1 = pl.pallas_call(
    _tc1_body,
    grid=(2, GRID),
    in_specs=[
        pl.BlockSpec((BN, D_IN), lambda c, i: (i, 0)),
        pl.BlockSpec((1, D_IN, HW1), lambda c, i: (c, 0, 0)),
        pl.BlockSpec((1, 1, HW1), lambda c, i: (c, 0, 0)),
    ],
    out_specs=pl.BlockSpec((1, BN, HW1), lambda c, i: (c, i, 0)),
    out_shape=jax.ShapeDtypeStruct((2, N_PAD, HW1), jnp.float32),
    name="sage_tc1",
)


def _tc2_body(x_ref, acc_ref, ws1_ref, b1_ref, ws2_ref, wn2_ref, b2_ref,
              p2_ref, hws_ref, invd_ref):
    dsum = acc_ref[1, :, D_H - HW1:D_H - HW1 + 1]
    inv = 1.0 / jnp.maximum(dsum, 1.0)
    invd_ref[...] = inv
    base = (jnp.dot(x_ref[...], ws1_ref[...],
                    preferred_element_type=jnp.float32) + b1_ref[...])
    h0 = jnp.maximum(base[:, 0:HW1] + acc_ref[0, :, :] * inv, 0.0)
    h1 = jnp.maximum(base[:, HW1:D_H]
                     + acc_ref[1, :, 0:D_H - HW1] * inv, 0.0)
    p2_ref[...] = (
        jnp.dot(h0, wn2_ref[0:HW1, :], preferred_element_type=jnp.float32)
        + jnp.dot(h1, wn2_ref[HW1:D_H, :], preferred_element_type=jnp.float32))
    hws_ref[...] = (
        jnp.dot(h0, ws2_ref[0:HW1, :], preferred_element_type=jnp.float32)
        + jnp.dot(h1, ws2_ref[HW1:D_H, :], preferred_element_type=jnp.float32)
        + b2_ref[...])


_tc2 = pl.pallas_call(
    _tc2_body,
    grid=(GRID,),
    in_specs=[
        pl.BlockSpec((BN, D_IN), lambda i: (i, 0)),
        pl.BlockSpec((2, BN, HW1), lambda i: (0, i, 0)),
        pl.BlockSpec((D_IN, D_H), lambda i: (0, 0)),
        pl.BlockSpec((1, D_H), lambda i: (0, 0)),
        pl.BlockSpec((D_H, D_OUT), lambda i: (0, 0)),
        pl.BlockSpec((D_H, D_OUT), lambda i: (0, 0)),
        pl.BlockSpec((1, D_OUT), lambda i: (0, 0)),
    ],
    out_specs=[
        pl.BlockSpec((BN, D_OUT), lambda i: (i, 0)),
        pl.BlockSpec((BN, D_OUT), lambda i: (i, 0)),
        pl.BlockSpec((BN, 1), lambda i: (i, 0)),
    ],
    out_shape=[
        jax.ShapeDtypeStruct((N_PAD, D_OUT), jnp.float32),
        jax.ShapeDtypeStruct((N_PAD, D_OUT), jnp.float32),
        jax.ShapeDtypeStruct((N_PAD, 1), jnp.float32),
    ],
    name="sage_tc2",
)


def _tc3_body(hws_ref, acc_ref, invd_ref, out_ref):
    a = acc_ref[0, :, :] + acc_ref[1, :, :]
    out_ref[...] = hws_ref[...] + a * invd_ref[...]


_tc3 = pl.pallas_call(
    _tc3_body,
    grid=(GRID,),
    in_specs=[
        pl.BlockSpec((BN, D_OUT), lambda i: (i, 0)),
        pl.BlockSpec((2, BN, D_OUT), lambda i: (0, i, 0)),
        pl.BlockSpec((BN, 1), lambda i: (i, 0)),
    ],
    out_specs=pl.BlockSpec((BN, D_OUT), lambda i: (i, 0)),
    out_shape=jax.ShapeDtypeStruct((N_PAD, D_OUT), jnp.float32),
    name="sage_tc3",
)


def kernel(x, edge_index, W_self1, W_neigh1, b1, W_self2, W_neigh2, b2):
    src = edge_index[0]
    dst = edge_index[1]
    pad = E_PAD - E
    src_p = jnp.concatenate([src, jnp.zeros((pad,), jnp.int32)]
                            ).reshape(-1, CH)
    dst_p = jnp.concatenate([dst, jnp.full((pad,), N, jnp.int32)]
                            ).reshape(-1, CH)
    x_p = jnp.pad(x, ((0, N_PAD - N), (0, 0)))
    wa0 = W_neigh1[:, 0:HW1]
    wa1 = jnp.zeros((D_IN, HW1), jnp.float32)
    wa1 = wa1.at[:, 0:D_H - HW1].set(W_neigh1[:, HW1:D_H])
    w_aug = jnp.stack([wa0, wa1])
    wrow = jnp.zeros((2, 1, HW1), jnp.float32).at[1, 0, D_H - HW1].set(1.0)

    z1 = jnp.zeros((RPS, HW1), jnp.float32)
    z2 = jnp.zeros((RPS, D_OUT), jnp.float32)

    t1 = _tc1(x_p, w_aug, wrow)
    acc1 = _segsum1(t1, src_p, dst_p, z1)
    p2, hws2, invd = _tc2(x_p, acc1, W_self1, b1.reshape(1, D_H),
                          W_self2, W_neigh2, b2.reshape(1, D_OUT))
    acc2 = _segsum2(p2, src_p, dst_p, z2)
    out = _tc3(hws2, acc2, invd)
    return out[:N]
